# Initial kernel scaffold; baseline (speedup 1.0000x reference)
import jax
import jax.numpy as jnp
from jax.experimental import pallas as pl


def kernel(pointcloud, W1, W2, W3, W4, g1, b1, g2, b2, g3, b3, g4, b4, mask1, mask2):
    raise NotImplementedError("write your pallas kernel here")


# speedup vs baseline: 2.4280x; 2.4280x over previous
import functools

import jax
import jax.numpy as jnp
from jax import lax
from jax.experimental import pallas as pl
from jax.experimental.pallas import tpu as pltpu
from jax.experimental.pallas import tpu_sc as plsc

B_, N_, K1_, K2_ = 8, 1024, 16, 4
BN_ = B_ * N_
EPS = 1e-5


def _topk_body(pc_ref, idx_ref):
    xt = pc_ref[0]
    inner = -2.0 * lax.dot_general(
        xt, xt, (((1,), (1,)), ((), ())), preferred_element_type=jnp.float32
    )
    sq = xt * xt
    xx = jnp.sum(sq, axis=1, keepdims=True)
    ones_row = jnp.ones((1, 8), jnp.float32)
    xx_row = lax.dot_general(
        ones_row, sq, (((1,), (1,)), ((), ())),
        preferred_element_type=jnp.float32,
        precision=lax.Precision.HIGHEST,
    )
    vals = -xx - inner - xx_row
    cols = lax.broadcasted_iota(jnp.int32, (N_, N_), 1)
    picks = []
    for _ in range(K1_):
        m = jnp.max(vals, axis=1, keepdims=True)
        cand = jnp.where(vals == m, cols, N_)
        amin = jnp.min(cand, axis=1, keepdims=True)
        picks.append(amin)
        vals = jnp.where(cols == amin, -jnp.inf, vals)
    idx_ref[0] = jnp.concatenate(picks, axis=1)


def _topk_call(pc_pad):
    return pl.pallas_call(
        _topk_body,
        grid=(B_,),
        in_specs=[pl.BlockSpec((1, N_, 8), lambda b: (b, 0, 0))],
        out_specs=pl.BlockSpec((1, N_, K1_), lambda b: (b, 0, 0)),
        out_shape=jax.ShapeDtypeStruct((B_, N_, K1_), jnp.int32),
    )(pc_pad)


def _sc_gather(table, idx_flat):
    info = plsc.get_sparse_core_info()
    nw = info.num_cores * info.num_subcores
    tot = idx_flat.shape[0]
    b_per_w = tot // nw
    mesh = plsc.VectorSubcoreMesh(core_axis_name="c", subcore_axis_name="s")

    @functools.partial(
        pl.kernel,
        mesh=mesh,
        compiler_params=pltpu.CompilerParams(use_tc_tiling_on_sc=False),
        out_type=jax.ShapeDtypeStruct((tot, 16), jnp.float32),
        scratch_types=[
            pltpu.VMEM((b_per_w,), jnp.int32),
            pltpu.VMEM((b_per_w, 16), jnp.float32),
            pltpu.SemaphoreType.DMA,
        ],
    )
    def gather_k(table_hbm, idx_hbm, out_hbm, idx_v, rows_v, sem):
        wid = lax.axis_index("s") * info.num_cores + lax.axis_index("c")
        base = wid * b_per_w
        pltpu.sync_copy(idx_hbm.at[pl.ds(base, b_per_w)], idx_v)
        pltpu.async_copy(table_hbm.at[idx_v], rows_v, sem).wait()
        pltpu.sync_copy(rows_v, out_hbm.at[pl.ds(base, b_per_w)])

    return gather_k(table, idx_flat)


MBLK = 1024
NJ = BN_ // MBLK


def _l1_body(x_ref, w_ref, y_ref, s_ref, q_ref, s4_ref, q4_ref):
    j = pl.program_id(0)
    k = pl.program_id(1)
    y = lax.dot_general(
        w_ref[...], x_ref[...], (((1,), (1,)), ((), ())),
        preferred_element_type=jnp.float32,
    )
    y_ref[...] = y
    s = jnp.sum(y, axis=1, keepdims=True)
    q = jnp.sum(y * y, axis=1, keepdims=True)

    @pl.when((j == 0) & (k == 0))
    def _():
        s_ref[...] = jnp.zeros_like(s_ref)
        q_ref[...] = jnp.zeros_like(q_ref)
        s4_ref[...] = jnp.zeros_like(s4_ref)
        q4_ref[...] = jnp.zeros_like(q4_ref)

    s_ref[...] += s
    q_ref[...] += q

    @pl.when(k < K2_)
    def _():
        s4_ref[...] += s
        q4_ref[...] += q


def _l1_call(rows, w_pad):
    m = K1_ * BN_
    cout = w_pad.shape[0]
    return pl.pallas_call(
        _l1_body,
        grid=(NJ, K1_),
        in_specs=[
            pl.BlockSpec((MBLK, 16), lambda j, k: (k * NJ + j, 0)),
            pl.BlockSpec((cout, 16), lambda j, k: (0, 0)),
        ],
        out_specs=[
            pl.BlockSpec((cout, MBLK), lambda j, k: (0, k * NJ + j)),
            pl.BlockSpec((cout, 128), lambda j, k: (0, 0)),
            pl.BlockSpec((cout, 128), lambda j, k: (0, 0)),
            pl.BlockSpec((cout, 128), lambda j, k: (0, 0)),
            pl.BlockSpec((cout, 128), lambda j, k: (0, 0)),
        ],
        out_shape=[
            jax.ShapeDtypeStruct((cout, m), jnp.float32),
            jax.ShapeDtypeStruct((cout, 128), jnp.float32),
            jax.ShapeDtypeStruct((cout, 128), jnp.float32),
            jax.ShapeDtypeStruct((cout, 128), jnp.float32),
            jax.ShapeDtypeStruct((cout, 128), jnp.float32),
        ],
    )(rows, w_pad)


def _mid_body(x_ref, sc_ref, sh_ref, w_ref, m_ref, y_ref, s_ref, q_ref, mx_ref):
    j = pl.program_id(0)
    k = pl.program_id(1)
    h = jnp.maximum(x_ref[...] * sc_ref[...] + sh_ref[...], 0.0)
    hm = h * m_ref[0]

    @pl.when(k == 0)
    def _():
        mx_ref[...] = hm

    @pl.when(k > 0)
    def _():
        mx_ref[...] = jnp.maximum(mx_ref[...], hm)

    y = jnp.dot(w_ref[...], h, preferred_element_type=jnp.float32)
    y_ref[...] = y
    s = jnp.sum(y, axis=1, keepdims=True)
    q = jnp.sum(y * y, axis=1, keepdims=True)

    @pl.when((j == 0) & (k == 0))
    def _():
        s_ref[...] = jnp.zeros_like(s_ref)
        q_ref[...] = jnp.zeros_like(q_ref)

    s_ref[...] += s
    q_ref[...] += q


def _mid_call(y_prev, scale, shift, w, mask3, kk):
    cin, m = y_prev.shape
    cout = w.shape[0]
    return pl.pallas_call(
        _mid_body,
        grid=(NJ, kk),
        in_specs=[
            pl.BlockSpec((cin, MBLK), lambda j, k: (0, k * NJ + j)),
            pl.BlockSpec((cin, 1), lambda j, k: (0, 0)),
            pl.BlockSpec((cin, 1), lambda j, k: (0, 0)),
            pl.BlockSpec((cout, cin), lambda j, k: (0, 0)),
            pl.BlockSpec((1, 1, MBLK), lambda j, k: (k, 0, j)),
        ],
        out_specs=[
            pl.BlockSpec((cout, MBLK), lambda j, k: (0, k * NJ + j)),
            pl.BlockSpec((cout, 128), lambda j, k: (0, 0)),
            pl.BlockSpec((cout, 128), lambda j, k: (0, 0)),
            pl.BlockSpec((cin, MBLK), lambda j, k: (0, j)),
        ],
        out_shape=[
            jax.ShapeDtypeStruct((cout, m), jnp.float32),
            jax.ShapeDtypeStruct((cout, 128), jnp.float32),
            jax.ShapeDtypeStruct((cout, 128), jnp.float32),
            jax.ShapeDtypeStruct((cin, BN_), jnp.float32),
        ],
    )(y_prev, scale, shift, w, mask3)


def _fin_body(x_ref, sc_ref, sh_ref, m_ref, mx_ref):
    k = pl.program_id(1)
    h = jnp.maximum(x_ref[...] * sc_ref[...] + sh_ref[...], 0.0)
    hm = h * m_ref[0]

    @pl.when(k == 0)
    def _():
        mx_ref[...] = hm

    @pl.when(k > 0)
    def _():
        mx_ref[...] = jnp.maximum(mx_ref[...], hm)


def _fin_call(y_prev, scale, shift, mask3, kk):
    cin, _ = y_prev.shape
    return pl.pallas_call(
        _fin_body,
        grid=(NJ, kk),
        in_specs=[
            pl.BlockSpec((cin, MBLK), lambda j, k: (0, k * NJ + j)),
            pl.BlockSpec((cin, 1), lambda j, k: (0, 0)),
            pl.BlockSpec((cin, 1), lambda j, k: (0, 0)),
            pl.BlockSpec((1, 1, MBLK), lambda j, k: (k, 0, j)),
        ],
        out_specs=pl.BlockSpec((cin, MBLK), lambda j, k: (0, j)),
        out_shape=jax.ShapeDtypeStruct((cin, BN_), jnp.float32),
    )(y_prev, scale, shift, mask3)


def _affine(s, q, g, b, count):
    mean = s[:, 0] / count
    var = q[:, 0] / count - mean * mean
    scale = g / jnp.sqrt(var + EPS)
    shift = b - mean * scale
    return scale[:, None], shift[:, None]


def _branch(y1, s1, q1, mask3, kk, W2, W3, W4, g1, b1, g2, b2, g3, b3, g4, b4):
    m = kk * BN_
    sc1, sh1 = _affine(s1, q1, g1, b1, m)
    y2, s2, q2, mx1 = _mid_call(y1, sc1, sh1, W2, mask3, kk)
    sc2, sh2 = _affine(s2, q2, g2, b2, m)
    y3, s3, q3, mx2 = _mid_call(y2, sc2, sh2, W3, mask3, kk)
    sc3, sh3 = _affine(s3, q3, g3, b3, m)
    y4, s4, q4, mx3 = _mid_call(y3, sc3, sh3, W4, mask3, kk)
    sc4, sh4 = _affine(s4, q4, g4, b4, m)
    mx4 = _fin_call(y4, sc4, sh4, mask3, kk)
    cat = jnp.concatenate([mx1, mx2, mx3, mx4], axis=0)
    return jnp.transpose(cat.reshape(512, B_, N_), (1, 0, 2))


def kernel(pointcloud, W1, W2, W3, W4, g1, b1, g2, b2, g3, b3, g4, b4, mask1, mask2):
    pc_pad = jnp.pad(pointcloud, ((0, 0), (0, 0), (0, 5)))
    idx = _topk_call(pc_pad)

    base = (jnp.arange(B_, dtype=jnp.int32) * N_)[:, None, None]
    gidx = jnp.transpose(idx + base, (2, 0, 1)).reshape(-1)
    table = jnp.pad(pointcloud.reshape(BN_, 3), ((0, 0), (0, 13)))
    rows = _sc_gather(table, gidx)

    w1p = jnp.pad(W1, ((0, 0), (0, 13)))
    y1, s1, q1, s1b, q1b = _l1_call(rows, w1p)

    m1 = jnp.transpose(mask1, (3, 1, 0, 2)).reshape(K1_, 1, BN_)
    m2 = jnp.transpose(mask2, (3, 1, 0, 2)).reshape(K2_, 1, BN_)

    lf1_m = _branch(y1, s1, q1, m1, K1_,
                    W2, W3, W4, g1, b1, g2, b2, g3, b3, g4, b4)
    y1b = y1[:, : K2_ * BN_]
    lf2_m = _branch(y1b, s1b, q1b, m2, K2_,
                    W2, W3, W4, g1, b1, g2, b2, g3, b3, g4, b4)
    return (lf1_m, lf2_m)


# speedup vs baseline: 2.8808x; 1.1865x over previous
import functools

import jax
import jax.numpy as jnp
from jax import lax
from jax.experimental import pallas as pl
from jax.experimental.pallas import tpu as pltpu
from jax.experimental.pallas import tpu_sc as plsc

B_, N_, K1_, K2_ = 8, 1024, 16, 4
BN_ = B_ * N_
EPS = 1e-5


def _topk_body(pc_ref, idx_ref):
    xt = pc_ref[0]
    inner = -2.0 * lax.dot_general(
        xt, xt, (((1,), (1,)), ((), ())), preferred_element_type=jnp.float32
    )
    sq = xt * xt
    xx = jnp.sum(sq, axis=1, keepdims=True)
    ones_row = jnp.ones((1, 8), jnp.float32)
    xx_row = lax.dot_general(
        ones_row, sq, (((1,), (1,)), ((), ())),
        preferred_element_type=jnp.float32,
        precision=lax.Precision.HIGHEST,
    )
    vals = -xx - inner - xx_row
    cols = lax.broadcasted_iota(jnp.int32, (N_, N_), 1)
    picks = []
    for _ in range(K1_):
        m = jnp.max(vals, axis=1, keepdims=True)
        cand = jnp.where(vals == m, cols, N_)
        amin = jnp.min(cand, axis=1, keepdims=True)
        picks.append(amin)
        vals = jnp.where(cols == amin, -jnp.inf, vals)
    idx_ref[0] = jnp.concatenate(picks, axis=1)


def _topk_call(pc_pad):
    return pl.pallas_call(
        _topk_body,
        grid=(B_,),
        in_specs=[pl.BlockSpec((1, N_, 8), lambda b: (b, 0, 0))],
        out_specs=pl.BlockSpec((1, N_, K1_), lambda b: (b, 0, 0)),
        out_shape=jax.ShapeDtypeStruct((B_, N_, K1_), jnp.int32),
    )(pc_pad)


def _sc_gather(table, idx_flat):
    info = plsc.get_sparse_core_info()
    nw = info.num_cores * info.num_subcores
    tot = idx_flat.shape[0]
    b_per_w = tot // nw
    mesh = plsc.VectorSubcoreMesh(core_axis_name="c", subcore_axis_name="s")

    @functools.partial(
        pl.kernel,
        mesh=mesh,
        compiler_params=pltpu.CompilerParams(use_tc_tiling_on_sc=False),
        out_type=jax.ShapeDtypeStruct((tot, 16), jnp.float32),
        scratch_types=[
            pltpu.VMEM((b_per_w,), jnp.int32),
            pltpu.VMEM((b_per_w, 16), jnp.float32),
            pltpu.SemaphoreType.DMA,
        ],
    )
    def gather_k(table_hbm, idx_hbm, out_hbm, idx_v, rows_v, sem):
        wid = lax.axis_index("s") * info.num_cores + lax.axis_index("c")
        base = wid * b_per_w
        pltpu.sync_copy(idx_hbm.at[pl.ds(base, b_per_w)], idx_v)
        pltpu.async_copy(table_hbm.at[idx_v], rows_v, sem).wait()
        pltpu.sync_copy(rows_v, out_hbm.at[pl.ds(base, b_per_w)])

    return gather_k(table, idx_flat)


MBLK = 2048
NJ = BN_ // MBLK


def _chain_body(kk, rows_ref, mask_ref, w1_ref, w2_ref, w3_ref, w4_ref,
                g1_ref, b1_ref, g2_ref, b2_ref, g3_ref, b3_ref, g4_ref, b4_ref,
                mx1_ref, mx2_ref, mx3_ref, mx4_ref,
                s1, q1, s2, q2, s3, q3, s4, q4):
    p = pl.program_id(0)
    j = pl.program_id(1)
    k = pl.program_id(2)
    cnt = float(kk * BN_)
    mk = mask_ref[0]

    @pl.when((p == 0) & (j == 0) & (k == 0))
    def _():
        for r in (s1, q1, s2, q2, s3, q3, s4, q4):
            r[...] = jnp.zeros_like(r)

    def acc(sref, qref, y):
        sref[...] += jnp.sum(y, axis=1, keepdims=True)
        qref[...] += jnp.sum(y * y, axis=1, keepdims=True)

    def aff_relu(y, sref, qref, gref, bref):
        mean = sref[:, :1] / cnt
        var = qref[:, :1] / cnt - mean * mean
        sc = gref[...] / jnp.sqrt(var + EPS)
        sh = bref[...] - mean * sc
        return jnp.maximum(y * sc + sh, 0.0)

    def upd_max(ref, h):
        hm = h * mk

        @pl.when(k == 0)
        def _():
            ref[...] = hm

        @pl.when(k > 0)
        def _():
            ref[...] = jnp.maximum(ref[...], hm)

    y1 = lax.dot_general(
        w1_ref[...], rows_ref[...], (((1,), (1,)), ((), ())),
        preferred_element_type=jnp.float32,
    )

    def h1_():
        return aff_relu(y1, s1, q1, g1_ref, b1_ref)

    def h2_(h1):
        y2 = jnp.dot(w2_ref[...], h1, preferred_element_type=jnp.float32)
        return aff_relu(y2, s2, q2, g2_ref, b2_ref)

    def h3_(h2):
        y3 = jnp.dot(w3_ref[...], h2, preferred_element_type=jnp.float32)
        return aff_relu(y3, s3, q3, g3_ref, b3_ref)

    @pl.when(p == 0)
    def _():
        acc(s1, q1, y1)

    @pl.when(p == 1)
    def _():
        h1 = h1_()
        upd_max(mx1_ref, h1)
        acc(s2, q2, jnp.dot(w2_ref[...], h1, preferred_element_type=jnp.float32))

    @pl.when(p == 2)
    def _():
        h2 = h2_(h1_())
        upd_max(mx2_ref, h2)
        acc(s3, q3, jnp.dot(w3_ref[...], h2, preferred_element_type=jnp.float32))

    @pl.when(p == 3)
    def _():
        h3 = h3_(h2_(h1_()))
        upd_max(mx3_ref, h3)
        acc(s4, q4, jnp.dot(w4_ref[...], h3, preferred_element_type=jnp.float32))

    @pl.when(p == 4)
    def _():
        y4 = jnp.dot(w4_ref[...], h3_(h2_(h1_())),
                     preferred_element_type=jnp.float32)
        h4 = aff_relu(y4, s4, q4, g4_ref, b4_ref)
        upd_max(mx4_ref, h4)


def _chain_call(rows, mask3, kk, w1p, W2, W3, W4,
                g1, b1, g2, b2, g3, b3, g4, b4):
    def park(active_p):
        return lambda p, j, k: (
            0, jnp.where(p == active_p, j, jnp.where(p < active_p, NJ, NJ + 1)))

    cdim = BN_ + 2 * MBLK
    out = pl.pallas_call(
        functools.partial(_chain_body, kk),
        grid=(5, NJ, kk),
        in_specs=[
            pl.BlockSpec((MBLK, 16), lambda p, j, k: (k * NJ + j, 0)),
            pl.BlockSpec((1, 1, MBLK), lambda p, j, k: (k, 0, j)),
        ] + [pl.BlockSpec(w.shape, lambda p, j, k: (0, 0))
             for w in (w1p, W2, W3, W4)]
          + [pl.BlockSpec((v.shape[0], 1), lambda p, j, k: (0, 0))
             for v in (g1, b1, g2, b2, g3, b3, g4, b4)],
        out_specs=[
            pl.BlockSpec((64, MBLK), park(1)),
            pl.BlockSpec((64, MBLK), park(2)),
            pl.BlockSpec((128, MBLK), park(3)),
            pl.BlockSpec((256, MBLK), park(4)),
        ],
        out_shape=[
            jax.ShapeDtypeStruct((64, cdim), jnp.float32),
            jax.ShapeDtypeStruct((64, cdim), jnp.float32),
            jax.ShapeDtypeStruct((128, cdim), jnp.float32),
            jax.ShapeDtypeStruct((256, cdim), jnp.float32),
        ],
        scratch_shapes=[
            pltpu.VMEM((64, 128), jnp.float32), pltpu.VMEM((64, 128), jnp.float32),
            pltpu.VMEM((64, 128), jnp.float32), pltpu.VMEM((64, 128), jnp.float32),
            pltpu.VMEM((128, 128), jnp.float32), pltpu.VMEM((128, 128), jnp.float32),
            pltpu.VMEM((256, 128), jnp.float32), pltpu.VMEM((256, 128), jnp.float32),
        ],
    )(rows, mask3, w1p, W2, W3, W4,
      g1[:, None], b1[:, None], g2[:, None], b2[:, None],
      g3[:, None], b3[:, None], g4[:, None], b4[:, None])
    mx1, mx2, mx3, mx4 = out
    cat = jnp.concatenate(
        [mx1[:, :BN_], mx2[:, :BN_], mx3[:, :BN_], mx4[:, :BN_]], axis=0)
    return jnp.transpose(cat.reshape(512, B_, N_), (1, 0, 2))


def kernel(pointcloud, W1, W2, W3, W4, g1, b1, g2, b2, g3, b3, g4, b4, mask1, mask2):
    pc_pad = jnp.pad(pointcloud, ((0, 0), (0, 0), (0, 5)))
    idx = _topk_call(pc_pad)

    base = (jnp.arange(B_, dtype=jnp.int32) * N_)[:, None, None]
    gidx = jnp.transpose(idx + base, (2, 0, 1)).reshape(-1)
    table = jnp.pad(pointcloud.reshape(BN_, 3), ((0, 0), (0, 13)))
    rows = _sc_gather(table, gidx)

    w1p = jnp.pad(W1, ((0, 0), (0, 13)))
    m1 = jnp.transpose(mask1, (3, 1, 0, 2)).reshape(K1_, 1, BN_)
    m2 = jnp.transpose(mask2, (3, 1, 0, 2)).reshape(K2_, 1, BN_)

    lf1_m = _chain_call(rows, m1, K1_, w1p, W2, W3, W4,
                        g1, b1, g2, b2, g3, b3, g4, b4)
    lf2_m = _chain_call(rows, m2, K2_, w1p, W2, W3, W4,
                        g1, b1, g2, b2, g3, b3, g4, b4)
    return (lf1_m, lf2_m)
